# baseline (speedup 1.0000x reference)
import functools

import jax
import jax.numpy as jnp
from jax import lax
from jax.experimental import pallas as pl
from jax.experimental.pallas import tpu as pltpu
from jax.experimental.pallas import tpu_sc as plsc

D = 768
NUM_EXPERTS = 8
ACTIVE = 2
TB = 2048
TOKENS = 32768
NW = 32
PW = TOKENS // NW
L = 16
GROUPS = PW // L


def _scores_block(x_ref, w_ref, st_ref):
    st_ref[...] = lax.dot_general(
        w_ref[...], x_ref[...],
        (((0,), (1,)), ((), ())),
        preferred_element_type=jnp.float32,
    )


def _scores_t(x, weights):
    tokens = x.shape[0]
    return pl.pallas_call(
        _scores_block,
        grid=(tokens // TB,),
        in_specs=[
            pl.BlockSpec((TB, D), lambda i: (i, 0)),
            pl.BlockSpec((D, NUM_EXPERTS), lambda i: (0, 0)),
        ],
        out_specs=pl.BlockSpec((NUM_EXPERTS, TB), lambda i: (0, i)),
        out_shape=jax.ShapeDtypeStruct((NUM_EXPERTS, tokens), jnp.float32),
    )(x, weights)


_mesh = plsc.VectorSubcoreMesh(core_axis_name="c", subcore_axis_name="s")


@functools.partial(
    pl.kernel,
    mesh=_mesh,
    out_type=[
        jax.ShapeDtypeStruct((ACTIVE, TOKENS), jnp.float32),
        jax.ShapeDtypeStruct((ACTIVE, TOKENS), jnp.int32),
    ],
    scratch_types=[
        pltpu.VMEM((NUM_EXPERTS, PW), jnp.float32),
        pltpu.VMEM((PW,), jnp.float32),
        pltpu.VMEM((PW,), jnp.float32),
        pltpu.VMEM((PW,), jnp.int32),
        pltpu.VMEM((PW,), jnp.int32),
    ],
)
def _route(st_hbm, ps_hbm, es_hbm, sv, p1v, p2v, i1v, i2v):
    wid = lax.axis_index("s") * 2 + lax.axis_index("c")
    base = wid * PW
    pltpu.sync_copy(st_hbm.at[:, pl.ds(base, PW)], sv)

    def body(g, carry):
        off = g * L
        m1 = sv[0, pl.ds(off, L)]
        i1 = jnp.zeros((L,), jnp.int32)
        m2 = jnp.full((L,), -jnp.inf, jnp.float32)
        i2 = jnp.zeros((L,), jnp.int32)
        for e in range(1, NUM_EXPERTS):
            se = sv[e, pl.ds(off, L)]
            ev = jnp.full((L,), e, jnp.int32)
            gt1 = se > m1
            gt2 = se > m2
            m2 = jnp.where(gt1, m1, jnp.where(gt2, se, m2))
            i2 = jnp.where(gt1, i1, jnp.where(gt2, ev, i2))
            m1 = jnp.where(gt1, se, m1)
            i1 = jnp.where(gt1, ev, i1)
        ex = jnp.exp(m2 - m1)
        den = 1.0 + ex
        p1v[pl.ds(off, L)] = 1.0 / den
        p2v[pl.ds(off, L)] = ex / den
        i1v[pl.ds(off, L)] = i1
        i2v[pl.ds(off, L)] = i2
        return carry

    lax.fori_loop(0, GROUPS, body, 0)
    pltpu.sync_copy(p1v, ps_hbm.at[0, pl.ds(base, PW)])
    pltpu.sync_copy(p2v, ps_hbm.at[1, pl.ds(base, PW)])
    pltpu.sync_copy(i1v, es_hbm.at[0, pl.ds(base, PW)])
    pltpu.sync_copy(i2v, es_hbm.at[1, pl.ds(base, PW)])


@jax.jit
def kernel(x, weights):
    st = _scores_t(x, weights)
    ps, es = _route(st)
    return (ps.T, es.T)


# speedup vs baseline: 1.4072x; 1.4072x over previous
import jax
import jax.numpy as jnp
from jax import lax
from jax.experimental import pallas as pl

D = 768
NUM_EXPERTS = 8
ACTIVE = 2
TB = 2048
LANES = 128
SUB = TB // LANES


def _gate_block(x_ref, w_ref, scores_ref, experts_ref):
    st = lax.dot_general(
        w_ref[...], x_ref[...],
        (((0,), (1,)), ((), ())),
        preferred_element_type=jnp.float32,
    )
    st3 = jnp.reshape(st, (NUM_EXPERTS, SUB, LANES))
    m1 = st3[0]
    i1 = jnp.zeros((SUB, LANES), jnp.int32)
    m2 = jnp.full((SUB, LANES), -jnp.inf, jnp.float32)
    i2 = jnp.zeros((SUB, LANES), jnp.int32)
    for e in range(1, NUM_EXPERTS):
        se = st3[e]
        ev = jnp.full((SUB, LANES), e, jnp.int32)
        gt1 = se > m1
        gt2 = se > m2
        m2 = jnp.where(gt1, m1, jnp.where(gt2, se, m2))
        i2 = jnp.where(gt1, i1, jnp.where(gt2, ev, i2))
        m1 = jnp.where(gt1, se, m1)
        i1 = jnp.where(gt1, ev, i1)
    ex = jnp.exp(m2 - m1)
    den = 1.0 + ex
    scores_ref[0] = 1.0 / den
    scores_ref[1] = ex / den
    experts_ref[0] = i1
    experts_ref[1] = i2


@jax.jit
def kernel(x, weights):
    tokens = x.shape[0]
    grid = (tokens // TB,)
    scores_t, experts_t = pl.pallas_call(
        _gate_block,
        grid=grid,
        in_specs=[
            pl.BlockSpec((TB, D), lambda i: (i, 0)),
            pl.BlockSpec((D, NUM_EXPERTS), lambda i: (0, 0)),
        ],
        out_specs=[
            pl.BlockSpec((ACTIVE, SUB, LANES), lambda i: (0, i, 0)),
            pl.BlockSpec((ACTIVE, SUB, LANES), lambda i: (0, i, 0)),
        ],
        out_shape=[
            jax.ShapeDtypeStruct((ACTIVE, tokens // LANES, LANES), jnp.float32),
            jax.ShapeDtypeStruct((ACTIVE, tokens // LANES, LANES), jnp.int32),
        ],
    )(x, weights)
    scores = jnp.reshape(scores_t, (ACTIVE, tokens)).T
    experts = jnp.reshape(experts_t, (ACTIVE, tokens)).T
    return (scores, experts)


# speedup vs baseline: 1.5336x; 1.0898x over previous
import jax
import jax.numpy as jnp
from jax.experimental import pallas as pl

D = 768
NUM_EXPERTS = 8
ACTIVE = 2
TB = 2048


def _gate_block(x_ref, w_ref, scores_ref, experts_ref):
    st = jax.lax.dot_general(
        w_ref[...], x_ref[...],
        (((0,), (1,)), ((), ())),
        preferred_element_type=jnp.float32,
    )
    row = jax.lax.broadcasted_iota(jnp.int32, st.shape, 0)
    m1 = jnp.max(st, axis=0, keepdims=True)
    i1 = jnp.min(jnp.where(st == m1, row, NUM_EXPERTS), axis=0, keepdims=True)
    masked = jnp.where(row == i1, -jnp.inf, st)
    m2 = jnp.max(masked, axis=0, keepdims=True)
    i2 = jnp.min(jnp.where(masked == m2, row, NUM_EXPERTS), axis=0, keepdims=True)
    e = jnp.exp(m2 - m1)
    denom = 1.0 + e
    scores_ref[...] = jnp.concatenate([1.0 / denom, e / denom], axis=0)
    experts_ref[...] = jnp.concatenate([i1, i2], axis=0)


@jax.jit
def kernel(x, weights):
    tokens = x.shape[0]
    grid = (tokens // TB,)
    scores_t, experts_t = pl.pallas_call(
        _gate_block,
        grid=grid,
        in_specs=[
            pl.BlockSpec((TB, D), lambda i: (i, 0)),
            pl.BlockSpec((D, NUM_EXPERTS), lambda i: (0, 0)),
        ],
        out_specs=[
            pl.BlockSpec((ACTIVE, TB), lambda i: (0, i)),
            pl.BlockSpec((ACTIVE, TB), lambda i: (0, i)),
        ],
        out_shape=[
            jax.ShapeDtypeStruct((ACTIVE, tokens), jnp.float32),
            jax.ShapeDtypeStruct((ACTIVE, tokens), jnp.int32),
        ],
    )(x, weights)
    return (scores_t.T, experts_t.T)


# speedup vs baseline: 1.5480x; 1.0094x over previous
import jax
import jax.numpy as jnp
from jax import lax
from jax.experimental import pallas as pl

D = 768
NUM_EXPERTS = 8
ACTIVE = 2
TB = 2048
HALF = TB // 2


def _route(st):
    row = lax.broadcasted_iota(jnp.int32, st.shape, 0)
    m1 = jnp.max(st, axis=0, keepdims=True)
    i1 = jnp.min(jnp.where(st == m1, row, NUM_EXPERTS), axis=0, keepdims=True)
    masked = jnp.where(row == i1, -jnp.inf, st)
    m2 = jnp.max(masked, axis=0, keepdims=True)
    i2 = jnp.min(jnp.where(masked == m2, row, NUM_EXPERTS), axis=0, keepdims=True)
    e = jnp.exp(m2 - m1)
    den = 1.0 + e
    return (
        jnp.concatenate([1.0 / den, e / den], axis=0),
        jnp.concatenate([i1, i2], axis=0),
    )


def _gate_block(x_ref, w_ref, scores_ref, experts_ref):
    for h in range(2):
        st = lax.dot_general(
            w_ref[...], x_ref[pl.ds(h * HALF, HALF), :],
            (((0,), (1,)), ((), ())),
            preferred_element_type=jnp.float32,
        )
        p, i = _route(st)
        scores_ref[:, pl.ds(h * HALF, HALF)] = p
        experts_ref[:, pl.ds(h * HALF, HALF)] = i


@jax.jit
def kernel(x, weights):
    tokens = x.shape[0]
    grid = (tokens // TB,)
    scores_t, experts_t = pl.pallas_call(
        _gate_block,
        grid=grid,
        in_specs=[
            pl.BlockSpec((TB, D), lambda i: (i, 0)),
            pl.BlockSpec((D, NUM_EXPERTS), lambda i: (0, 0)),
        ],
        out_specs=[
            pl.BlockSpec((ACTIVE, TB), lambda i: (0, i)),
            pl.BlockSpec((ACTIVE, TB), lambda i: (0, i)),
        ],
        out_shape=[
            jax.ShapeDtypeStruct((ACTIVE, tokens), jnp.float32),
            jax.ShapeDtypeStruct((ACTIVE, tokens), jnp.int32),
        ],
    )(x, weights)
    return (scores_t.T, experts_t.T)
